# baseline (speedup 1.0000x reference)
import functools

import jax
import jax.numpy as jnp
import numpy as np
from jax import lax
from jax.experimental import pallas as pl
from jax.experimental.pallas import tpu as pltpu
from jax.experimental.pallas import tpu_sc as plsc

N = 10000
E = 320000
D = 128
H = 32
OUT = 128

NC, NS, L = 2, 16, 16
NW = NC * NS
EB = 1280
NBLK = E // EB
KMAX = -(-NBLK // NW)
GS = 128
NBG = EB // GS

ACCN = 10240
EB3 = 6400
NB3 = E // EB3

NEG = -3.0e38


@functools.lru_cache(maxsize=None)
def _sc_mesh():
    return plsc.VectorSubcoreMesh(core_axis_name="c", subcore_axis_name="s")


def _f32(shape):
    return jax.ShapeDtypeStruct(shape, jnp.float32)


NPAD = 10240


def _a2_body(x_ref, w_ref, c_ref, o_ref):
    o_ref[:, :] = (
        lax.dot_general(
            w_ref[:, :], x_ref[:, :], (((0,), (1,)), ((), ())),
            preferred_element_type=jnp.float32,
        )
        + c_ref[:, :]
    )


def _node_fold(xp, w1s, cvect):
    return pl.pallas_call(
        _a2_body,
        grid=(10,),
        in_specs=[
            pl.BlockSpec((1024, D), lambda i: (i, 0)),
            pl.BlockSpec((D, H), lambda i: (0, 0)),
            pl.BlockSpec((H, 1), lambda i: (0, 0)),
        ],
        out_specs=pl.BlockSpec((H, 1024), lambda i: (0, i)),
        out_shape=_f32((H, NPAD)),
    )(xp, w1s, cvect)


SB = 6400
NSB = E // SB


@functools.lru_cache(maxsize=None)
def _edge_front_kernel():
    return pl.kernel(
        _edge_front_body,
        mesh=_sc_mesh(),
        out_type=(_f32((6 * E,)), _f32((NW * E,))),
        scratch_types=[
            pltpu.VMEM((6 * N,), jnp.float32),
            pltpu.VMEM((EB,), jnp.int32),
            pltpu.VMEM((EB,), jnp.int32),
            pltpu.VMEM((6 * EB,), jnp.float32),
            pltpu.VMEM((NPAD,), jnp.float32),
            pltpu.VMEM((SB,), jnp.int32),
            pltpu.VMEM((SB,), jnp.float32),
        ],
        compiler_params=pltpu.CompilerParams(needs_layout_passes=False),
    )


def _edge_front_body(t6_hbm, src_hbm, dst_hbm, a2t_hbm,
                     dots_out, a2g_out, t6v, srcv, dstv, dotsv,
                     tabv, sbuf, obuf):
    wid = lax.axis_index("s") * NC + lax.axis_index("c")
    pltpu.sync_copy(t6_hbm, t6v)

    def block(k, carry):
        blk = wid + NW * k

        @pl.when(blk < NBLK)
        def _():
            base = blk * EB
            pltpu.sync_copy(src_hbm.at[pl.ds(base, EB)], srcv)
            pltpu.sync_copy(dst_hbm.at[pl.ds(base, EB)], dstv)

            def vec(j, c2):
                s16 = srcv[pl.ds(j * 16, 16)]
                d16 = dstv[pl.ds(j * 16, 16)]
                pjx = plsc.load_gather(t6v, [s16])
                pjy = plsc.load_gather(t6v, [s16 + N])
                pjz = plsc.load_gather(t6v, [s16 + 2 * N])
                njx = plsc.load_gather(t6v, [s16 + 3 * N])
                njy = plsc.load_gather(t6v, [s16 + 4 * N])
                njz = plsc.load_gather(t6v, [s16 + 5 * N])
                pix = plsc.load_gather(t6v, [d16])
                piy = plsc.load_gather(t6v, [d16 + N])
                piz = plsc.load_gather(t6v, [d16 + 2 * N])
                nix = plsc.load_gather(t6v, [d16 + 3 * N])
                niy = plsc.load_gather(t6v, [d16 + 4 * N])
                niz = plsc.load_gather(t6v, [d16 + 5 * N])
                psx = pjx - pix
                psy = pjy - piy
                psz = pjz - piz
                dd = psx * psx + psy * psy + psz * psz
                nips = nix * psx + niy * psy + niz * psz
                njps = njx * psx + njy * psy + njz * psz
                ninj = nix * njx + niy * njy + niz * njz
                nini = nix * nix + niy * niy + niz * niz
                njnj = njx * njx + njy * njy + njz * njz
                for f, v in enumerate((dd, nips, njps, ninj, nini, njnj)):
                    dotsv[pl.ds(f * EB + j * 16, 16)] = v
                return c2

            lax.fori_loop(0, EB // 16, vec, 0, unroll=2)
            for f in range(6):
                pltpu.sync_copy(
                    dotsv.at[pl.ds(f * EB, EB)],
                    dots_out.at[pl.ds(f * E + base, EB)],
                )

        return carry

    lax.fori_loop(0, KMAX, block, 0)

    pltpu.sync_copy(a2t_hbm.at[pl.ds(wid * NPAD, NPAD)], tabv)

    def gblock(b, carry):
        pltpu.sync_copy(src_hbm.at[pl.ds(b * SB, SB)], sbuf)

        def gvec(j, c2):
            obuf[pl.ds(j * 16, 16)] = plsc.load_gather(
                tabv, [sbuf[pl.ds(j * 16, 16)]]
            )
            return c2

        lax.fori_loop(0, SB // 16, gvec, 0, unroll=8)
        pltpu.sync_copy(obuf, a2g_out.at[pl.ds(wid * E + b * SB, SB)])
        return carry

    lax.fori_loop(0, NSB, gblock, 0)


def _atan2_poly(y, x):
    ax = jnp.abs(x)
    hi = jnp.maximum(y, ax)
    lo = jnp.minimum(y, ax)
    r = lo / jnp.maximum(hi, jnp.float32(1e-37))
    s = r * r
    p = jnp.float32(2.90188402868554315e-03)
    p = p * s - jnp.float32(1.62980136087791103e-02)
    p = p * s + jnp.float32(4.30512745506233115e-02)
    p = p * s - jnp.float32(7.53037674892936771e-02)
    p = p * s + jnp.float32(1.06554379229240167e-01)
    p = p * s - jnp.float32(1.42077862045390988e-01)
    p = p * s + jnp.float32(1.99926389418588319e-01)
    p = p * s - jnp.float32(3.33331017604993073e-01)
    a = r + r * s * p
    a = jnp.where(y > ax, jnp.float32(np.pi / 2) - a, a)
    a = jnp.where(x < 0, jnp.float32(np.pi) - a, a)
    return jnp.where((y == 0) & (x == 0), jnp.float32(0.0), a)


EB2 = 6400


def _edge_mlp_body(a2gt_ref, dots_ref, w1pp_ref, w2_ref, o_ref):
    dots = dots_ref[:, :]
    dd = dots[0:1, :]
    nips = dots[1:2, :]
    njps = dots[2:3, :]
    ninj = dots[3:4, :]
    nini = dots[4:5, :]
    njnj = dots[5:6, :]
    relu = lambda v: jnp.maximum(v, jnp.float32(0.0))
    d = jnp.sqrt(dd)
    a1 = _atan2_poly(jnp.sqrt(relu(nini * dd - nips * nips)), nips)
    a2 = _atan2_poly(jnp.sqrt(relu(njnj * dd - njps * njps)), njps)
    a3 = _atan2_poly(jnp.sqrt(relu(nini * njnj - ninj * ninj)), ninj)
    ppf = jnp.concatenate([d, a1, a2, a3], axis=0)
    pre = a2gt_ref[:, :] + lax.dot_general(
        w1pp_ref[:, :], ppf, (((0,), (0,)), ((), ())),
        preferred_element_type=jnp.float32,
    )
    t = relu(pre)
    o_ref[:, :] = lax.dot_general(
        w2_ref[:, :], t, (((0,), (0,)), ((), ())),
        preferred_element_type=jnp.float32,
    )


def _edge_mlp(a2gt, dots, w1pp, w2):
    return pl.pallas_call(
        _edge_mlp_body,
        grid=(E // EB2,),
        in_specs=[
            pl.BlockSpec((H, EB2), lambda i: (0, i)),
            pl.BlockSpec((6, EB2), lambda i: (0, i)),
            pl.BlockSpec((4, H), lambda i: (0, 0)),
            pl.BlockSpec((H, H), lambda i: (0, 0)),
        ],
        out_specs=pl.BlockSpec((H, EB2), lambda i: (0, i)),
        out_shape=_f32((H, E)),
    )(a2gt, dots, w1pp, w2)


@functools.lru_cache(maxsize=None)
def _segment_max_kernel():
    return pl.kernel(
        _segment_max_body,
        mesh=_sc_mesh(),
        out_type=_f32((NW * ACCN,)),
        scratch_types=[
            pltpu.VMEM((ACCN,), jnp.float32),
            pltpu.VMEM((EB3,), jnp.int32),
            pltpu.VMEM((EB3,), jnp.float32),
            pltpu.VMEM((EB3,), jnp.int32),
            pltpu.VMEM((EB3,), jnp.float32),
            pltpu.SemaphoreType.DMA,
            pltpu.SemaphoreType.DMA,
        ],
        compiler_params=pltpu.CompilerParams(needs_layout_passes=False),
    )


def _segment_max_body(dst_hbm, ht_hbm, m_out, acc, dA, vA, dB, vB,
                      semA, semB):
    col = lax.axis_index("s") * NC + lax.axis_index("c")
    negv = jnp.full((16,), NEG, jnp.float32)

    def init(i, c):
        acc[pl.ds(i * 16, 16)] = negv
        return c

    lax.fori_loop(0, ACCN // 16, init, 0)

    def start(k, dref, vref, sem):
        pltpu.async_copy(dst_hbm.at[pl.ds(k * EB3, EB3)], dref, sem)
        pltpu.async_copy(ht_hbm.at[pl.ds(col * E + k * EB3, EB3)], vref, sem)

    def drain(dref, vref, sem):
        pltpu.make_async_copy(dst_hbm.at[pl.ds(0, EB3)], dref, sem).wait()
        pltpu.make_async_copy(ht_hbm.at[pl.ds(0, EB3)], vref, sem).wait()

    def process(dref, vref):
        def vec(j, c2):
            idx = dref[pl.ds(j * 16, 16)]
            val = vref[pl.ds(j * 16, 16)]
            a = plsc.load_gather(acc, [idx])
            cur = jnp.maximum(a, val)
            plsc.store_scatter(acc, [idx], cur)
            g = plsc.load_gather(acc, [idx])
            lost = g < cur
            ncnt = plsc.all_reduce_population_count(lost)

            @pl.when(ncnt[0] > 0)
            def _():
                def cond(pend):
                    return jnp.any(pend)

                def body(pend):
                    plsc.store_scatter(acc, [idx], cur, mask=pend)
                    g2 = plsc.load_gather(acc, [idx])
                    return pend & (g2 < cur)

                lax.while_loop(cond, body, lost)

            return c2

        lax.fori_loop(0, EB3 // 16, vec, 0, unroll=4)

    start(0, dA, vA, semA)

    def pair(p, carry):
        start(2 * p + 1, dB, vB, semB)
        drain(dA, vA, semA)
        process(dA, vA)

        @pl.when(p < NB3 // 2 - 1)
        def _():
            start(2 * p + 2, dA, vA, semA)

        drain(dB, vB, semB)
        process(dB, vB)
        return carry

    lax.fori_loop(0, NB3 // 2, pair, 0)
    pltpu.sync_copy(acc, m_out.at[pl.ds(col * ACCN, ACCN)])


CB = 1024


def _out_mlp_body(mt_ref, b2_ref, w3_ref, b3_ref, g2_ref, be2_ref,
                  w4_ref, b4_ref, z_ref, st_ref, sc):
    i = pl.program_id(0)
    relu = lambda v: jnp.maximum(v, jnp.float32(0.0))
    m = mt_ref[:, :].T
    aggr = jnp.where(m > jnp.float32(-1e38), m + b2_ref[:, :], jnp.float32(0.0))
    z1 = relu(
        (jnp.dot(aggr, w3_ref[:, :], preferred_element_type=jnp.float32)
         + b3_ref[:, :]) * g2_ref[:, :] + be2_ref[:, :]
    )
    z = relu(
        jnp.dot(z1, w4_ref[:, :], preferred_element_type=jnp.float32)
        + b4_ref[:, :]
    )
    rows = i * CB + lax.broadcasted_iota(jnp.int32, (CB, 1), 0)
    mask = rows < N
    bm = jnp.max(jnp.where(mask, z, jnp.float32(NEG)))

    @pl.when(i == 0)
    def _():
        sc[0] = jnp.float32(NEG)
        sc[1] = jnp.float32(0.0)

    mold = sc[0]
    mn = jnp.maximum(mold, bm)
    s = sc[1] * jnp.exp(mold - mn) + jnp.sum(
        jnp.where(mask, jnp.exp(z - mn), jnp.float32(0.0))
    )
    sc[0] = mn
    sc[1] = s

    @pl.when(i == (ACCN // CB) - 1)
    def _():
        st_ref[:, :] = jnp.concatenate(
            [jnp.broadcast_to(mn, (1, 1)), jnp.broadcast_to(s, (1, 1))], axis=1
        )

    z_ref[:, :] = z


def _out_mlp(mt, b2, w3, b3, g2v, be2, w4, b4):
    return pl.pallas_call(
        _out_mlp_body,
        grid=(ACCN // CB,),
        in_specs=[
            pl.BlockSpec((H, CB), lambda i: (0, i)),
            pl.BlockSpec((1, H), lambda i: (0, 0)),
            pl.BlockSpec((H, H), lambda i: (0, 0)),
            pl.BlockSpec((1, H), lambda i: (0, 0)),
            pl.BlockSpec((1, H), lambda i: (0, 0)),
            pl.BlockSpec((1, H), lambda i: (0, 0)),
            pl.BlockSpec((H, OUT), lambda i: (0, 0)),
            pl.BlockSpec((1, OUT), lambda i: (0, 0)),
        ],
        out_specs=[
            pl.BlockSpec((CB, OUT), lambda i: (i, 0)),
            pl.BlockSpec((1, 2), lambda i: (0, 0)),
        ],
        out_shape=[_f32((ACCN, OUT)), _f32((1, 2))],
        scratch_shapes=[pltpu.SMEM((2,), jnp.float32)],
    )(mt, b2, w3, b3, g2v, be2, w4, b4)


def _norm_body(z_ref, st_ref, y_ref):
    st = st_ref[:, :]
    y_ref[:, :] = jnp.exp(z_ref[:, :] - st[0:1, 0:1]) * (
        jnp.float32(1.0) / st[0:1, 1:2]
    )


def _normalize(z10, stats):
    return pl.pallas_call(
        _norm_body,
        grid=(25,),
        in_specs=[
            pl.BlockSpec((400, OUT), lambda i: (i, 0)),
            pl.BlockSpec((1, 2), lambda i: (0, 0)),
        ],
        out_specs=pl.BlockSpec((400, OUT), lambda i: (i, 0)),
        out_shape=_f32((N, OUT)),
    )(z10, stats)


def kernel(x, pos, norm, edge_index, W1, b1, g1, be1, W2, b2, W3, b3, g2,
           be2, W4, b4):
    src = edge_index[0]
    dst = edge_index[1]
    w1s = W1[:D] * g1[None, :]
    cvect = (g1 * b1 + be1)[:, None]
    w1pp = W1[D:] * g1[None, :]

    t6 = jnp.concatenate(
        [pos[:, 0], pos[:, 1], pos[:, 2], norm[:, 0], norm[:, 1], norm[:, 2]]
    )
    xp = jnp.pad(x, ((0, NPAD - N), (0, 0)))
    a2t = _node_fold(xp, w1s, cvect)
    dotsf, a2gf = _edge_front_kernel()(t6, src, dst, a2t.reshape(-1))
    ht = _edge_mlp(a2gf.reshape(NW, E), dotsf.reshape(6, E), w1pp, W2)
    mflat = _segment_max_kernel()(dst, ht.reshape(-1))
    mt = mflat.reshape(NW, ACCN)
    z, stats = _out_mlp(mt, b2[None, :], W3, b3[None, :], g2[None, :],
                        be2[None, :], W4, b4[None, :])
    y = _normalize(z[:N], stats)
    return y.reshape(-1)


# speedup vs baseline: 1.0153x; 1.0153x over previous
import functools

import jax
import jax.numpy as jnp
import numpy as np
from jax import lax
from jax.experimental import pallas as pl
from jax.experimental.pallas import tpu as pltpu
from jax.experimental.pallas import tpu_sc as plsc

N = 10000
E = 320000
D = 128
H = 32
OUT = 128

NC, NS, L = 2, 16, 16
NW = NC * NS
EB = 1280
NBLK = E // EB
KMAX = -(-NBLK // NW)
GS = 128
NBG = EB // GS

ACCN = 10240
EB3 = 6400
NB3 = E // EB3

NEG = -3.0e38


@functools.lru_cache(maxsize=None)
def _sc_mesh():
    return plsc.VectorSubcoreMesh(core_axis_name="c", subcore_axis_name="s")


def _f32(shape):
    return jax.ShapeDtypeStruct(shape, jnp.float32)


NPAD = 10240


def _a2_body(x_ref, w_ref, c_ref, o_ref):
    o_ref[:, :] = (
        lax.dot_general(
            w_ref[:, :], x_ref[:, :], (((0,), (1,)), ((), ())),
            preferred_element_type=jnp.float32,
        )
        + c_ref[:, :]
    )


def _node_fold(xp, w1s, cvect):
    return pl.pallas_call(
        _a2_body,
        grid=(10,),
        in_specs=[
            pl.BlockSpec((1024, D), lambda i: (i, 0)),
            pl.BlockSpec((D, H), lambda i: (0, 0)),
            pl.BlockSpec((H, 1), lambda i: (0, 0)),
        ],
        out_specs=pl.BlockSpec((H, 1024), lambda i: (0, i)),
        out_shape=_f32((H, NPAD)),
    )(xp, w1s, cvect)


SB = 6400
NSB = E // SB


@functools.lru_cache(maxsize=None)
def _edge_front_kernel():
    return pl.kernel(
        _edge_front_body,
        mesh=_sc_mesh(),
        out_type=(_f32((6 * E,)), _f32((NW * E,))),
        scratch_types=[
            pltpu.VMEM((6 * N,), jnp.float32),
            pltpu.VMEM((EB,), jnp.int32),
            pltpu.VMEM((EB,), jnp.int32),
            pltpu.VMEM((6 * EB,), jnp.float32),
            pltpu.VMEM((NPAD,), jnp.float32),
            pltpu.VMEM((SB,), jnp.int32),
            pltpu.VMEM((SB,), jnp.float32),
        ],
        compiler_params=pltpu.CompilerParams(needs_layout_passes=False),
    )


def _edge_front_body(t6_hbm, src_hbm, dst_hbm, a2t_hbm,
                     dots_out, a2g_out, t6v, srcv, dstv, dotsv,
                     tabv, sbuf, obuf):
    wid = lax.axis_index("s") * NC + lax.axis_index("c")
    pltpu.sync_copy(t6_hbm, t6v)

    def block(k, carry):
        blk = wid + NW * k

        @pl.when(blk < NBLK)
        def _():
            base = blk * EB
            pltpu.sync_copy(src_hbm.at[pl.ds(base, EB)], srcv)
            pltpu.sync_copy(dst_hbm.at[pl.ds(base, EB)], dstv)

            def vec(j, c2):
                s16 = srcv[pl.ds(j * 16, 16)]
                d16 = dstv[pl.ds(j * 16, 16)]
                pjx = plsc.load_gather(t6v, [s16])
                pjy = plsc.load_gather(t6v, [s16 + N])
                pjz = plsc.load_gather(t6v, [s16 + 2 * N])
                njx = plsc.load_gather(t6v, [s16 + 3 * N])
                njy = plsc.load_gather(t6v, [s16 + 4 * N])
                njz = plsc.load_gather(t6v, [s16 + 5 * N])
                pix = plsc.load_gather(t6v, [d16])
                piy = plsc.load_gather(t6v, [d16 + N])
                piz = plsc.load_gather(t6v, [d16 + 2 * N])
                nix = plsc.load_gather(t6v, [d16 + 3 * N])
                niy = plsc.load_gather(t6v, [d16 + 4 * N])
                niz = plsc.load_gather(t6v, [d16 + 5 * N])
                psx = pjx - pix
                psy = pjy - piy
                psz = pjz - piz
                dd = psx * psx + psy * psy + psz * psz
                nips = nix * psx + niy * psy + niz * psz
                njps = njx * psx + njy * psy + njz * psz
                ninj = nix * njx + niy * njy + niz * njz
                nini = nix * nix + niy * niy + niz * niz
                njnj = njx * njx + njy * njy + njz * njz
                for f, v in enumerate((dd, nips, njps, ninj, nini, njnj)):
                    dotsv[pl.ds(f * EB + j * 16, 16)] = v
                return c2

            lax.fori_loop(0, EB // 16, vec, 0, unroll=2)
            for f in range(6):
                pltpu.sync_copy(
                    dotsv.at[pl.ds(f * EB, EB)],
                    dots_out.at[pl.ds(f * E + base, EB)],
                )

        return carry

    lax.fori_loop(0, KMAX, block, 0)

    pltpu.sync_copy(a2t_hbm.at[pl.ds(wid * NPAD, NPAD)], tabv)

    def gblock(b, carry):
        pltpu.sync_copy(src_hbm.at[pl.ds(b * SB, SB)], sbuf)

        def gvec(j, c2):
            obuf[pl.ds(j * 16, 16)] = plsc.load_gather(
                tabv, [sbuf[pl.ds(j * 16, 16)]]
            )
            return c2

        lax.fori_loop(0, SB // 16, gvec, 0, unroll=8)
        pltpu.sync_copy(obuf, a2g_out.at[pl.ds(wid * E + b * SB, SB)])
        return carry

    lax.fori_loop(0, NSB, gblock, 0)


def _atan2_poly(y, x):
    ax = jnp.abs(x)
    hi = jnp.maximum(y, ax)
    lo = jnp.minimum(y, ax)
    r = lo / jnp.maximum(hi, jnp.float32(1e-37))
    s = r * r
    p = jnp.float32(2.90188402868554315e-03)
    p = p * s - jnp.float32(1.62980136087791103e-02)
    p = p * s + jnp.float32(4.30512745506233115e-02)
    p = p * s - jnp.float32(7.53037674892936771e-02)
    p = p * s + jnp.float32(1.06554379229240167e-01)
    p = p * s - jnp.float32(1.42077862045390988e-01)
    p = p * s + jnp.float32(1.99926389418588319e-01)
    p = p * s - jnp.float32(3.33331017604993073e-01)
    a = r + r * s * p
    a = jnp.where(y > ax, jnp.float32(np.pi / 2) - a, a)
    a = jnp.where(x < 0, jnp.float32(np.pi) - a, a)
    return jnp.where((y == 0) & (x == 0), jnp.float32(0.0), a)


EB2 = 6400


def _edge_mlp_body(a2gt_ref, dots_ref, w1pp_ref, w2_ref, o_ref):
    dots = dots_ref[:, :]
    dd = dots[0:1, :]
    nips = dots[1:2, :]
    njps = dots[2:3, :]
    ninj = dots[3:4, :]
    nini = dots[4:5, :]
    njnj = dots[5:6, :]
    relu = lambda v: jnp.maximum(v, jnp.float32(0.0))
    d = jnp.sqrt(dd)
    a1 = _atan2_poly(jnp.sqrt(relu(nini * dd - nips * nips)), nips)
    a2 = _atan2_poly(jnp.sqrt(relu(njnj * dd - njps * njps)), njps)
    a3 = _atan2_poly(jnp.sqrt(relu(nini * njnj - ninj * ninj)), ninj)
    ppf = jnp.concatenate([d, a1, a2, a3], axis=0)
    pre = a2gt_ref[:, :] + lax.dot_general(
        w1pp_ref[:, :], ppf, (((0,), (0,)), ((), ())),
        preferred_element_type=jnp.float32,
    )
    t = relu(pre)
    o_ref[:, :] = lax.dot_general(
        w2_ref[:, :], t, (((0,), (0,)), ((), ())),
        preferred_element_type=jnp.float32,
    )


def _edge_mlp(a2gt, dots, w1pp, w2):
    return pl.pallas_call(
        _edge_mlp_body,
        grid=(E // EB2,),
        in_specs=[
            pl.BlockSpec((H, EB2), lambda i: (0, i)),
            pl.BlockSpec((6, EB2), lambda i: (0, i)),
            pl.BlockSpec((4, H), lambda i: (0, 0)),
            pl.BlockSpec((H, H), lambda i: (0, 0)),
        ],
        out_specs=pl.BlockSpec((H, EB2), lambda i: (0, i)),
        out_shape=_f32((H, E)),
    )(a2gt, dots, w1pp, w2)


@functools.lru_cache(maxsize=None)
def _segment_max_kernel():
    return pl.kernel(
        _segment_max_body,
        mesh=_sc_mesh(),
        out_type=_f32((NW * ACCN,)),
        scratch_types=[
            pltpu.VMEM((ACCN,), jnp.float32),
            pltpu.VMEM((ACCN,), jnp.int32),
            pltpu.VMEM((EB3,), jnp.int32),
            pltpu.VMEM((EB3,), jnp.float32),
            pltpu.VMEM((EB3,), jnp.int32),
            pltpu.VMEM((EB3,), jnp.float32),
            pltpu.SemaphoreType.DMA,
            pltpu.SemaphoreType.DMA,
        ],
        compiler_params=pltpu.CompilerParams(needs_layout_passes=False),
    )


def _segment_max_body(dst_hbm, ht_hbm, m_out, acc, mk, dA, vA, dB, vB,
                      semA, semB):
    col = lax.axis_index("s") * NC + lax.axis_index("c")
    lane = lax.iota(jnp.int32, 16)
    negv = jnp.full((16,), NEG, jnp.float32)

    def init(i, c):
        acc[pl.ds(i * 16, 16)] = negv
        return c

    lax.fori_loop(0, ACCN // 16, init, 0)

    def start(k, dref, vref, sem):
        pltpu.async_copy(dst_hbm.at[pl.ds(k * EB3, EB3)], dref, sem)
        pltpu.async_copy(ht_hbm.at[pl.ds(col * E + k * EB3, EB3)], vref, sem)

    def drain(dref, vref, sem):
        pltpu.make_async_copy(dst_hbm.at[pl.ds(0, EB3)], dref, sem).wait()
        pltpu.make_async_copy(ht_hbm.at[pl.ds(0, EB3)], vref, sem).wait()

    def process(dref, vref):
        def vec(j, c2):
            idx = dref[pl.ds(j * 16, 16)]
            val = vref[pl.ds(j * 16, 16)]
            plsc.store_scatter(mk, [idx], lane)
            g = plsc.load_gather(mk, [idx])
            win = g == lane
            a = plsc.load_gather(acc, [idx])
            plsc.store_scatter(acc, [idx], jnp.maximum(a, val), mask=win)
            act = jnp.logical_not(win)
            ncnt = plsc.all_reduce_population_count(act)

            @pl.when(ncnt[0] > 0)
            def _():
                def cond(pend):
                    return jnp.any(pend)

                def body(pend):
                    plsc.store_scatter(mk, [idx], lane, mask=pend)
                    g2 = plsc.load_gather(mk, [idx])
                    w2 = pend & (g2 == lane)
                    a2 = plsc.load_gather(acc, [idx])
                    plsc.store_scatter(
                        acc, [idx], jnp.maximum(a2, val), mask=w2
                    )
                    return pend & jnp.logical_not(w2)

                lax.while_loop(cond, body, act)

            return c2

        lax.fori_loop(0, EB3 // 16, vec, 0, unroll=2)

    start(0, dA, vA, semA)

    def pair(p, carry):
        start(2 * p + 1, dB, vB, semB)
        drain(dA, vA, semA)
        process(dA, vA)

        @pl.when(p < NB3 // 2 - 1)
        def _():
            start(2 * p + 2, dA, vA, semA)

        drain(dB, vB, semB)
        process(dB, vB)
        return carry

    lax.fori_loop(0, NB3 // 2, pair, 0)
    pltpu.sync_copy(acc, m_out.at[pl.ds(col * ACCN, ACCN)])


CB = 1024


def _out_mlp_body(mt_ref, b2_ref, w3_ref, b3_ref, g2_ref, be2_ref,
                  w4_ref, b4_ref, z_ref, st_ref, sc):
    i = pl.program_id(0)
    relu = lambda v: jnp.maximum(v, jnp.float32(0.0))
    m = mt_ref[:, :].T
    aggr = jnp.where(m > jnp.float32(-1e38), m + b2_ref[:, :], jnp.float32(0.0))
    z1 = relu(
        (jnp.dot(aggr, w3_ref[:, :], preferred_element_type=jnp.float32)
         + b3_ref[:, :]) * g2_ref[:, :] + be2_ref[:, :]
    )
    z = relu(
        jnp.dot(z1, w4_ref[:, :], preferred_element_type=jnp.float32)
        + b4_ref[:, :]
    )
    rows = i * CB + lax.broadcasted_iota(jnp.int32, (CB, 1), 0)
    mask = rows < N
    bm = jnp.max(jnp.where(mask, z, jnp.float32(NEG)))

    @pl.when(i == 0)
    def _():
        sc[0] = jnp.float32(NEG)
        sc[1] = jnp.float32(0.0)

    mold = sc[0]
    mn = jnp.maximum(mold, bm)
    s = sc[1] * jnp.exp(mold - mn) + jnp.sum(
        jnp.where(mask, jnp.exp(z - mn), jnp.float32(0.0))
    )
    sc[0] = mn
    sc[1] = s

    @pl.when(i == (ACCN // CB) - 1)
    def _():
        st_ref[:, :] = jnp.concatenate(
            [jnp.broadcast_to(mn, (1, 1)), jnp.broadcast_to(s, (1, 1))], axis=1
        )

    z_ref[:, :] = z


def _out_mlp(mt, b2, w3, b3, g2v, be2, w4, b4):
    return pl.pallas_call(
        _out_mlp_body,
        grid=(ACCN // CB,),
        in_specs=[
            pl.BlockSpec((H, CB), lambda i: (0, i)),
            pl.BlockSpec((1, H), lambda i: (0, 0)),
            pl.BlockSpec((H, H), lambda i: (0, 0)),
            pl.BlockSpec((1, H), lambda i: (0, 0)),
            pl.BlockSpec((1, H), lambda i: (0, 0)),
            pl.BlockSpec((1, H), lambda i: (0, 0)),
            pl.BlockSpec((H, OUT), lambda i: (0, 0)),
            pl.BlockSpec((1, OUT), lambda i: (0, 0)),
        ],
        out_specs=[
            pl.BlockSpec((CB, OUT), lambda i: (i, 0)),
            pl.BlockSpec((1, 2), lambda i: (0, 0)),
        ],
        out_shape=[_f32((ACCN, OUT)), _f32((1, 2))],
        scratch_shapes=[pltpu.SMEM((2,), jnp.float32)],
    )(mt, b2, w3, b3, g2v, be2, w4, b4)


def _norm_body(z_ref, st_ref, y_ref):
    st = st_ref[:, :]
    y_ref[:, :] = jnp.exp(z_ref[:, :] - st[0:1, 0:1]) * (
        jnp.float32(1.0) / st[0:1, 1:2]
    )


def _normalize(z10, stats):
    return pl.pallas_call(
        _norm_body,
        grid=(25,),
        in_specs=[
            pl.BlockSpec((400, OUT), lambda i: (i, 0)),
            pl.BlockSpec((1, 2), lambda i: (0, 0)),
        ],
        out_specs=pl.BlockSpec((400, OUT), lambda i: (i, 0)),
        out_shape=_f32((N, OUT)),
    )(z10, stats)


def kernel(x, pos, norm, edge_index, W1, b1, g1, be1, W2, b2, W3, b3, g2,
           be2, W4, b4):
    src = edge_index[0]
    dst = edge_index[1]
    w1s = W1[:D] * g1[None, :]
    cvect = (g1 * b1 + be1)[:, None]
    w1pp = W1[D:] * g1[None, :]

    t6 = jnp.concatenate(
        [pos[:, 0], pos[:, 1], pos[:, 2], norm[:, 0], norm[:, 1], norm[:, 2]]
    )
    xp = jnp.pad(x, ((0, NPAD - N), (0, 0)))
    a2t = _node_fold(xp, w1s, cvect)
    dotsf, a2gf = _edge_front_kernel()(t6, src, dst, a2t.reshape(-1))
    ht = _edge_mlp(a2gf.reshape(NW, E), dotsf.reshape(6, E), w1pp, W2)
    mflat = _segment_max_kernel()(dst, ht.reshape(-1))
    mt = mflat.reshape(NW, ACCN)
    z, stats = _out_mlp(mt, b2[None, :], W3, b3[None, :], g2[None, :],
                        be2[None, :], W4, b4[None, :])
    y = _normalize(z[:N], stats)
    return y.reshape(-1)


# speedup vs baseline: 1.0262x; 1.0108x over previous
import functools

import jax
import jax.numpy as jnp
import numpy as np
from jax import lax
from jax.experimental import pallas as pl
from jax.experimental.pallas import tpu as pltpu
from jax.experimental.pallas import tpu_sc as plsc

N = 10000
E = 320000
D = 128
H = 32
OUT = 128

NC, NS, L = 2, 16, 16
NW = NC * NS
EB = 1280
NBLK = E // EB
KMAX = -(-NBLK // NW)
GS = 128
NBG = EB // GS

ACCN = 10240
EB3 = 6400
NB3 = E // EB3

NEG = -3.0e38


@functools.lru_cache(maxsize=None)
def _sc_mesh():
    return plsc.VectorSubcoreMesh(core_axis_name="c", subcore_axis_name="s")


def _f32(shape):
    return jax.ShapeDtypeStruct(shape, jnp.float32)


NPAD = 10240


def _a2_body(x_ref, w_ref, c_ref, o_ref):
    o_ref[:, :] = (
        lax.dot_general(
            w_ref[:, :], x_ref[:, :], (((0,), (1,)), ((), ())),
            preferred_element_type=jnp.float32,
        )
        + c_ref[:, :]
    )


def _node_fold(xp, w1s, cvect):
    return pl.pallas_call(
        _a2_body,
        grid=(10,),
        in_specs=[
            pl.BlockSpec((1024, D), lambda i: (i, 0)),
            pl.BlockSpec((D, H), lambda i: (0, 0)),
            pl.BlockSpec((H, 1), lambda i: (0, 0)),
        ],
        out_specs=pl.BlockSpec((H, 1024), lambda i: (0, i)),
        out_shape=_f32((H, NPAD)),
    )(xp, w1s, cvect)


SB = 6400
NSB = E // SB


@functools.lru_cache(maxsize=None)
def _edge_front_kernel():
    return pl.kernel(
        _edge_front_body,
        mesh=_sc_mesh(),
        out_type=(_f32((6 * E,)), _f32((NW * E,))),
        scratch_types=[
            pltpu.VMEM((6 * N,), jnp.float32),
            pltpu.VMEM((EB,), jnp.int32),
            pltpu.VMEM((EB,), jnp.int32),
            pltpu.VMEM((6 * EB,), jnp.float32),
            pltpu.VMEM((NPAD,), jnp.float32),
            pltpu.VMEM((SB,), jnp.int32),
            pltpu.VMEM((SB,), jnp.float32),
        ],
        compiler_params=pltpu.CompilerParams(needs_layout_passes=False),
    )


def _edge_front_body(t6_hbm, src_hbm, dst_hbm, a2t_hbm,
                     dots_out, a2g_out, t6v, srcv, dstv, dotsv,
                     tabv, sbuf, obuf):
    wid = lax.axis_index("s") * NC + lax.axis_index("c")
    pltpu.sync_copy(t6_hbm, t6v)

    def block(k, carry):
        blk = wid + NW * k

        @pl.when(blk < NBLK)
        def _():
            base = blk * EB
            pltpu.sync_copy(src_hbm.at[pl.ds(base, EB)], srcv)
            pltpu.sync_copy(dst_hbm.at[pl.ds(base, EB)], dstv)

            def vec(j, c2):
                s16 = srcv[pl.ds(j * 16, 16)]
                d16 = dstv[pl.ds(j * 16, 16)]
                pjx = plsc.load_gather(t6v, [s16])
                pjy = plsc.load_gather(t6v, [s16 + N])
                pjz = plsc.load_gather(t6v, [s16 + 2 * N])
                njx = plsc.load_gather(t6v, [s16 + 3 * N])
                njy = plsc.load_gather(t6v, [s16 + 4 * N])
                njz = plsc.load_gather(t6v, [s16 + 5 * N])
                pix = plsc.load_gather(t6v, [d16])
                piy = plsc.load_gather(t6v, [d16 + N])
                piz = plsc.load_gather(t6v, [d16 + 2 * N])
                nix = plsc.load_gather(t6v, [d16 + 3 * N])
                niy = plsc.load_gather(t6v, [d16 + 4 * N])
                niz = plsc.load_gather(t6v, [d16 + 5 * N])
                psx = pjx - pix
                psy = pjy - piy
                psz = pjz - piz
                dd = psx * psx + psy * psy + psz * psz
                nips = nix * psx + niy * psy + niz * psz
                njps = njx * psx + njy * psy + njz * psz
                ninj = nix * njx + niy * njy + niz * njz
                nini = nix * nix + niy * niy + niz * niz
                njnj = njx * njx + njy * njy + njz * njz
                for f, v in enumerate((dd, nips, njps, ninj, nini, njnj)):
                    dotsv[pl.ds(f * EB + j * 16, 16)] = v
                return c2

            lax.fori_loop(0, EB // 16, vec, 0)
            for f in range(6):
                pltpu.sync_copy(
                    dotsv.at[pl.ds(f * EB, EB)],
                    dots_out.at[pl.ds(f * E + base, EB)],
                )

        return carry

    lax.fori_loop(0, KMAX, block, 0)

    pltpu.sync_copy(a2t_hbm.at[pl.ds(wid * NPAD, NPAD)], tabv)

    def gblock(b, carry):
        pltpu.sync_copy(src_hbm.at[pl.ds(b * SB, SB)], sbuf)

        def gvec(j, c2):
            obuf[pl.ds(j * 16, 16)] = plsc.load_gather(
                tabv, [sbuf[pl.ds(j * 16, 16)]]
            )
            return c2

        lax.fori_loop(0, SB // 16, gvec, 0)
        pltpu.sync_copy(obuf, a2g_out.at[pl.ds(wid * E + b * SB, SB)])
        return carry

    lax.fori_loop(0, NSB, gblock, 0)


def _atan2_poly(y, x):
    ax = jnp.abs(x)
    hi = jnp.maximum(y, ax)
    lo = jnp.minimum(y, ax)
    r = lo / jnp.maximum(hi, jnp.float32(1e-37))
    s = r * r
    p = jnp.float32(2.90188402868554315e-03)
    p = p * s - jnp.float32(1.62980136087791103e-02)
    p = p * s + jnp.float32(4.30512745506233115e-02)
    p = p * s - jnp.float32(7.53037674892936771e-02)
    p = p * s + jnp.float32(1.06554379229240167e-01)
    p = p * s - jnp.float32(1.42077862045390988e-01)
    p = p * s + jnp.float32(1.99926389418588319e-01)
    p = p * s - jnp.float32(3.33331017604993073e-01)
    a = r + r * s * p
    a = jnp.where(y > ax, jnp.float32(np.pi / 2) - a, a)
    a = jnp.where(x < 0, jnp.float32(np.pi) - a, a)
    return jnp.where((y == 0) & (x == 0), jnp.float32(0.0), a)


EB2 = 6400


def _edge_mlp_body(a2gt_ref, dots_ref, w1pp_ref, w2_ref, o_ref):
    dots = dots_ref[:, :]
    dd = dots[0:1, :]
    nips = dots[1:2, :]
    njps = dots[2:3, :]
    ninj = dots[3:4, :]
    nini = dots[4:5, :]
    njnj = dots[5:6, :]
    relu = lambda v: jnp.maximum(v, jnp.float32(0.0))
    d = jnp.sqrt(dd)
    a1 = _atan2_poly(jnp.sqrt(relu(nini * dd - nips * nips)), nips)
    a2 = _atan2_poly(jnp.sqrt(relu(njnj * dd - njps * njps)), njps)
    a3 = _atan2_poly(jnp.sqrt(relu(nini * njnj - ninj * ninj)), ninj)
    ppf = jnp.concatenate([d, a1, a2, a3], axis=0)
    pre = a2gt_ref[:, :] + lax.dot_general(
        w1pp_ref[:, :], ppf, (((0,), (0,)), ((), ())),
        preferred_element_type=jnp.float32,
    )
    t = relu(pre)
    o_ref[:, :] = lax.dot_general(
        w2_ref[:, :], t, (((0,), (0,)), ((), ())),
        preferred_element_type=jnp.float32,
    )


def _edge_mlp(a2gt, dots, w1pp, w2):
    return pl.pallas_call(
        _edge_mlp_body,
        grid=(E // EB2,),
        in_specs=[
            pl.BlockSpec((H, EB2), lambda i: (0, i)),
            pl.BlockSpec((6, EB2), lambda i: (0, i)),
            pl.BlockSpec((4, H), lambda i: (0, 0)),
            pl.BlockSpec((H, H), lambda i: (0, 0)),
        ],
        out_specs=pl.BlockSpec((H, EB2), lambda i: (0, i)),
        out_shape=_f32((H, E)),
    )(a2gt, dots, w1pp, w2)


@functools.lru_cache(maxsize=None)
def _segment_max_kernel():
    return pl.kernel(
        _segment_max_body,
        mesh=_sc_mesh(),
        out_type=_f32((NW * ACCN,)),
        scratch_types=[
            pltpu.VMEM((ACCN,), jnp.float32),
            pltpu.VMEM((ACCN,), jnp.int32),
            pltpu.VMEM((EB3,), jnp.int32),
            pltpu.VMEM((EB3,), jnp.float32),
            pltpu.VMEM((EB3,), jnp.int32),
            pltpu.VMEM((EB3,), jnp.float32),
            pltpu.SemaphoreType.DMA,
            pltpu.SemaphoreType.DMA,
        ],
        compiler_params=pltpu.CompilerParams(needs_layout_passes=False),
    )


def _segment_max_body(dst_hbm, ht_hbm, m_out, acc, mk, dA, vA, dB, vB,
                      semA, semB):
    col = lax.axis_index("s") * NC + lax.axis_index("c")
    lane = lax.iota(jnp.int32, 16)
    negv = jnp.full((16,), NEG, jnp.float32)

    def init(i, c):
        acc[pl.ds(i * 16, 16)] = negv
        return c

    lax.fori_loop(0, ACCN // 16, init, 0)

    def start(k, dref, vref, sem):
        pltpu.async_copy(dst_hbm.at[pl.ds(k * EB3, EB3)], dref, sem)
        pltpu.async_copy(ht_hbm.at[pl.ds(col * E + k * EB3, EB3)], vref, sem)

    def drain(dref, vref, sem):
        pltpu.make_async_copy(dst_hbm.at[pl.ds(0, EB3)], dref, sem).wait()
        pltpu.make_async_copy(ht_hbm.at[pl.ds(0, EB3)], vref, sem).wait()

    def process(dref, vref):
        def vec(j, c2):
            idx = dref[pl.ds(j * 16, 16)]
            val = vref[pl.ds(j * 16, 16)]
            plsc.store_scatter(mk, [idx], lane)
            g = plsc.load_gather(mk, [idx])
            win = g == lane
            a = plsc.load_gather(acc, [idx])
            plsc.store_scatter(acc, [idx], jnp.maximum(a, val), mask=win)
            act = jnp.logical_not(win)
            ncnt = plsc.all_reduce_population_count(act)

            @pl.when(ncnt[0] > 0)
            def _():
                def cond(pend):
                    return jnp.any(pend)

                def body(pend):
                    plsc.store_scatter(mk, [idx], lane, mask=pend)
                    g2 = plsc.load_gather(mk, [idx])
                    w2 = pend & (g2 == lane)
                    a2 = plsc.load_gather(acc, [idx])
                    plsc.store_scatter(
                        acc, [idx], jnp.maximum(a2, val), mask=w2
                    )
                    return pend & jnp.logical_not(w2)

                lax.while_loop(cond, body, act)

            return c2

        lax.fori_loop(0, EB3 // 16, vec, 0)

    start(0, dA, vA, semA)

    def pair(p, carry):
        start(2 * p + 1, dB, vB, semB)
        drain(dA, vA, semA)
        process(dA, vA)

        @pl.when(p < NB3 // 2 - 1)
        def _():
            start(2 * p + 2, dA, vA, semA)

        drain(dB, vB, semB)
        process(dB, vB)
        return carry

    lax.fori_loop(0, NB3 // 2, pair, 0)
    pltpu.sync_copy(acc, m_out.at[pl.ds(col * ACCN, ACCN)])


CB = 1024


def _out_mlp_body(mt_ref, b2_ref, w3_ref, b3_ref, g2_ref, be2_ref,
                  w4_ref, b4_ref, z_ref, st_ref, sc):
    i = pl.program_id(0)
    relu = lambda v: jnp.maximum(v, jnp.float32(0.0))
    m = mt_ref[:, :].T
    aggr = jnp.where(m > jnp.float32(-1e38), m + b2_ref[:, :], jnp.float32(0.0))
    z1 = relu(
        (jnp.dot(aggr, w3_ref[:, :], preferred_element_type=jnp.float32)
         + b3_ref[:, :]) * g2_ref[:, :] + be2_ref[:, :]
    )
    z = relu(
        jnp.dot(z1, w4_ref[:, :], preferred_element_type=jnp.float32)
        + b4_ref[:, :]
    )
    rows = i * CB + lax.broadcasted_iota(jnp.int32, (CB, 1), 0)
    mask = rows < N
    bm = jnp.max(jnp.where(mask, z, jnp.float32(NEG)))

    @pl.when(i == 0)
    def _():
        sc[0] = jnp.float32(NEG)
        sc[1] = jnp.float32(0.0)

    mold = sc[0]
    mn = jnp.maximum(mold, bm)
    s = sc[1] * jnp.exp(mold - mn) + jnp.sum(
        jnp.where(mask, jnp.exp(z - mn), jnp.float32(0.0))
    )
    sc[0] = mn
    sc[1] = s

    @pl.when(i == (ACCN // CB) - 1)
    def _():
        st_ref[:, :] = jnp.concatenate(
            [jnp.broadcast_to(mn, (1, 1)), jnp.broadcast_to(s, (1, 1))], axis=1
        )

    z_ref[:, :] = z


def _out_mlp(mt, b2, w3, b3, g2v, be2, w4, b4):
    return pl.pallas_call(
        _out_mlp_body,
        grid=(ACCN // CB,),
        in_specs=[
            pl.BlockSpec((H, CB), lambda i: (0, i)),
            pl.BlockSpec((1, H), lambda i: (0, 0)),
            pl.BlockSpec((H, H), lambda i: (0, 0)),
            pl.BlockSpec((1, H), lambda i: (0, 0)),
            pl.BlockSpec((1, H), lambda i: (0, 0)),
            pl.BlockSpec((1, H), lambda i: (0, 0)),
            pl.BlockSpec((H, OUT), lambda i: (0, 0)),
            pl.BlockSpec((1, OUT), lambda i: (0, 0)),
        ],
        out_specs=[
            pl.BlockSpec((CB, OUT), lambda i: (i, 0)),
            pl.BlockSpec((1, 2), lambda i: (0, 0)),
        ],
        out_shape=[_f32((ACCN, OUT)), _f32((1, 2))],
        scratch_shapes=[pltpu.SMEM((2,), jnp.float32)],
    )(mt, b2, w3, b3, g2v, be2, w4, b4)


def _norm_body(z_ref, st_ref, y_ref):
    st = st_ref[:, :]
    y_ref[:, :] = jnp.exp(z_ref[:, :] - st[0:1, 0:1]) * (
        jnp.float32(1.0) / st[0:1, 1:2]
    )


def _normalize(z10, stats):
    return pl.pallas_call(
        _norm_body,
        grid=(25,),
        in_specs=[
            pl.BlockSpec((400, OUT), lambda i: (i, 0)),
            pl.BlockSpec((1, 2), lambda i: (0, 0)),
        ],
        out_specs=pl.BlockSpec((400, OUT), lambda i: (i, 0)),
        out_shape=_f32((N, OUT)),
    )(z10, stats)


def kernel(x, pos, norm, edge_index, W1, b1, g1, be1, W2, b2, W3, b3, g2,
           be2, W4, b4):
    src = edge_index[0]
    dst = edge_index[1]
    w1s = W1[:D] * g1[None, :]
    cvect = (g1 * b1 + be1)[:, None]
    w1pp = W1[D:] * g1[None, :]

    t6 = jnp.concatenate(
        [pos[:, 0], pos[:, 1], pos[:, 2], norm[:, 0], norm[:, 1], norm[:, 2]]
    )
    xp = jnp.pad(x, ((0, NPAD - N), (0, 0)))
    a2t = _node_fold(xp, w1s, cvect)
    dotsf, a2gf = _edge_front_kernel()(t6, src, dst, a2t.reshape(-1))
    ht = _edge_mlp(a2gf.reshape(NW, E), dotsf.reshape(6, E), w1pp, W2)
    mflat = _segment_max_kernel()(dst, ht.reshape(-1))
    mt = mflat.reshape(NW, ACCN)
    z, stats = _out_mlp(mt, b2[None, :], W3, b3[None, :], g2[None, :],
                        be2[None, :], W4, b4[None, :])
    y = _normalize(z[:N], stats)
    return y.reshape(-1)


# speedup vs baseline: 2.1052x; 2.0514x over previous
import functools

import jax
import jax.numpy as jnp
import numpy as np
from jax import lax
from jax.experimental import pallas as pl
from jax.experimental.pallas import tpu as pltpu
from jax.experimental.pallas import tpu_sc as plsc

N = 10000
E = 320000
D = 128
H = 32
OUT = 128

NC, NS, L = 2, 16, 16
NW = NC * NS
EB = 1280
NBLK = E // EB
KMAX = -(-NBLK // NW)
GS = 128
NBG = EB // GS

ACCN = 10240
SPLITF = 4
NQ = NW // (H // SPLITF)
EQ = E // NQ
EB3 = 3200
NB3 = EQ // EB3

NEG = -3.0e38


@functools.lru_cache(maxsize=None)
def _sc_mesh():
    return plsc.VectorSubcoreMesh(core_axis_name="c", subcore_axis_name="s")


def _f32(shape):
    return jax.ShapeDtypeStruct(shape, jnp.float32)


NPAD = 10240


def _a2_body(x_ref, w_ref, c_ref, o_ref):
    o_ref[:, :] = (
        lax.dot_general(
            w_ref[:, :], x_ref[:, :], (((0,), (1,)), ((), ())),
            preferred_element_type=jnp.float32,
        )
        + c_ref[:, :]
    )


def _node_fold(xp, w1s, cvect):
    return pl.pallas_call(
        _a2_body,
        grid=(10,),
        in_specs=[
            pl.BlockSpec((1024, D), lambda i: (i, 0)),
            pl.BlockSpec((D, H), lambda i: (0, 0)),
            pl.BlockSpec((H, 1), lambda i: (0, 0)),
        ],
        out_specs=pl.BlockSpec((H, 1024), lambda i: (0, i)),
        out_shape=_f32((H, NPAD)),
    )(xp, w1s, cvect)


SB = 6400
NSB = E // SB


@functools.lru_cache(maxsize=None)
def _edge_front_kernel():
    return pl.kernel(
        _edge_front_body,
        mesh=_sc_mesh(),
        out_type=(_f32((6 * E,)), _f32((NW * E,))),
        scratch_types=[
            pltpu.VMEM((6 * N,), jnp.float32),
            pltpu.VMEM((EB,), jnp.int32),
            pltpu.VMEM((EB,), jnp.int32),
            pltpu.VMEM((6 * EB,), jnp.float32),
            pltpu.VMEM((NPAD,), jnp.float32),
            pltpu.VMEM((SB,), jnp.int32),
            pltpu.VMEM((SB,), jnp.float32),
        ],
        compiler_params=pltpu.CompilerParams(needs_layout_passes=False),
    )


def _edge_front_body(t6_hbm, src_hbm, dst_hbm, a2t_hbm,
                     dots_out, a2g_out, t6v, srcv, dstv, dotsv,
                     tabv, sbuf, obuf):
    wid = lax.axis_index("s") * NC + lax.axis_index("c")
    pltpu.sync_copy(t6_hbm, t6v)

    def block(k, carry):
        blk = wid + NW * k

        @pl.when(blk < NBLK)
        def _():
            base = blk * EB
            pltpu.sync_copy(src_hbm.at[pl.ds(base, EB)], srcv)
            pltpu.sync_copy(dst_hbm.at[pl.ds(base, EB)], dstv)

            def vec(j, c2):
                s16 = srcv[pl.ds(j * 16, 16)]
                d16 = dstv[pl.ds(j * 16, 16)]
                pjx = plsc.load_gather(t6v, [s16])
                pjy = plsc.load_gather(t6v, [s16 + N])
                pjz = plsc.load_gather(t6v, [s16 + 2 * N])
                njx = plsc.load_gather(t6v, [s16 + 3 * N])
                njy = plsc.load_gather(t6v, [s16 + 4 * N])
                njz = plsc.load_gather(t6v, [s16 + 5 * N])
                pix = plsc.load_gather(t6v, [d16])
                piy = plsc.load_gather(t6v, [d16 + N])
                piz = plsc.load_gather(t6v, [d16 + 2 * N])
                nix = plsc.load_gather(t6v, [d16 + 3 * N])
                niy = plsc.load_gather(t6v, [d16 + 4 * N])
                niz = plsc.load_gather(t6v, [d16 + 5 * N])
                psx = pjx - pix
                psy = pjy - piy
                psz = pjz - piz
                dd = psx * psx + psy * psy + psz * psz
                nips = nix * psx + niy * psy + niz * psz
                njps = njx * psx + njy * psy + njz * psz
                ninj = nix * njx + niy * njy + niz * njz
                nini = nix * nix + niy * niy + niz * niz
                njnj = njx * njx + njy * njy + njz * njz
                for f, v in enumerate((dd, nips, njps, ninj, nini, njnj)):
                    dotsv[pl.ds(f * EB + j * 16, 16)] = v
                return c2

            lax.fori_loop(0, EB // 16, vec, 0)
            for f in range(6):
                pltpu.sync_copy(
                    dotsv.at[pl.ds(f * EB, EB)],
                    dots_out.at[pl.ds(f * E + base, EB)],
                )

        return carry

    lax.fori_loop(0, KMAX, block, 0)

    pltpu.sync_copy(a2t_hbm.at[pl.ds(wid * NPAD, NPAD)], tabv)

    def gblock(b, carry):
        pltpu.sync_copy(src_hbm.at[pl.ds(b * SB, SB)], sbuf)

        def gvec(j, c2):
            obuf[pl.ds(j * 16, 16)] = plsc.load_gather(
                tabv, [sbuf[pl.ds(j * 16, 16)]]
            )
            return c2

        lax.fori_loop(0, SB // 16, gvec, 0)
        pltpu.sync_copy(obuf, a2g_out.at[pl.ds(wid * E + b * SB, SB)])
        return carry

    lax.fori_loop(0, NSB, gblock, 0)


def _atan2_poly(y, x):
    ax = jnp.abs(x)
    hi = jnp.maximum(y, ax)
    lo = jnp.minimum(y, ax)
    r = lo / jnp.maximum(hi, jnp.float32(1e-37))
    s = r * r
    p = jnp.float32(2.90188402868554315e-03)
    p = p * s - jnp.float32(1.62980136087791103e-02)
    p = p * s + jnp.float32(4.30512745506233115e-02)
    p = p * s - jnp.float32(7.53037674892936771e-02)
    p = p * s + jnp.float32(1.06554379229240167e-01)
    p = p * s - jnp.float32(1.42077862045390988e-01)
    p = p * s + jnp.float32(1.99926389418588319e-01)
    p = p * s - jnp.float32(3.33331017604993073e-01)
    a = r + r * s * p
    a = jnp.where(y > ax, jnp.float32(np.pi / 2) - a, a)
    a = jnp.where(x < 0, jnp.float32(np.pi) - a, a)
    return jnp.where((y == 0) & (x == 0), jnp.float32(0.0), a)


EB2 = 6400


def _edge_mlp_body(a2gt_ref, dots_ref, w1pp_ref, w2_ref, o_ref):
    dots = dots_ref[:, :]
    dd = dots[0:1, :]
    nips = dots[1:2, :]
    njps = dots[2:3, :]
    ninj = dots[3:4, :]
    nini = dots[4:5, :]
    njnj = dots[5:6, :]
    relu = lambda v: jnp.maximum(v, jnp.float32(0.0))
    d = jnp.sqrt(dd)
    a1 = _atan2_poly(jnp.sqrt(relu(nini * dd - nips * nips)), nips)
    a2 = _atan2_poly(jnp.sqrt(relu(njnj * dd - njps * njps)), njps)
    a3 = _atan2_poly(jnp.sqrt(relu(nini * njnj - ninj * ninj)), ninj)
    ppf = jnp.concatenate([d, a1, a2, a3], axis=0)
    pre = a2gt_ref[:, :] + lax.dot_general(
        w1pp_ref[:, :], ppf, (((0,), (0,)), ((), ())),
        preferred_element_type=jnp.float32,
    )
    t = relu(pre)
    o_ref[:, :] = lax.dot_general(
        w2_ref[:, :], t, (((0,), (0,)), ((), ())),
        preferred_element_type=jnp.float32,
    )


def _edge_mlp(a2gt, dots, w1pp, w2):
    return pl.pallas_call(
        _edge_mlp_body,
        grid=(E // EB2,),
        in_specs=[
            pl.BlockSpec((H, EB2), lambda i: (0, i)),
            pl.BlockSpec((6, EB2), lambda i: (0, i)),
            pl.BlockSpec((4, H), lambda i: (0, 0)),
            pl.BlockSpec((H, H), lambda i: (0, 0)),
        ],
        out_specs=pl.BlockSpec((H, EB2), lambda i: (0, i)),
        out_shape=_f32((H, E)),
    )(a2gt, dots, w1pp, w2)


@functools.lru_cache(maxsize=None)
def _segment_max_kernel():
    return pl.kernel(
        _segment_max_body,
        mesh=_sc_mesh(),
        out_type=_f32((NQ * NW * ACCN,)),
        scratch_types=[
            [pltpu.VMEM((ACCN,), jnp.float32) for _ in range(SPLITF)],
            pltpu.VMEM((ACCN,), jnp.int32),
            pltpu.VMEM((EB3,), jnp.int32),
            [pltpu.VMEM((EB3,), jnp.float32) for _ in range(SPLITF)],
            pltpu.VMEM((EB3,), jnp.int32),
            [pltpu.VMEM((EB3,), jnp.float32) for _ in range(SPLITF)],
            pltpu.SemaphoreType.DMA,
            pltpu.SemaphoreType.DMA,
        ],
        compiler_params=pltpu.CompilerParams(needs_layout_passes=False),
    )


def _segment_max_body(dst_hbm, ht_hbm, m_out, accs, mk, dA, vAs, dB, vBs,
                      semA, semB):
    wid = lax.axis_index("s") * NC + lax.axis_index("c")
    q = wid // (H // SPLITF)
    cp = wid % (H // SPLITF)
    ebase = q * EQ
    lane = lax.iota(jnp.int32, 16)
    negv = jnp.full((16,), NEG, jnp.float32)

    def init(i, c):
        for ac in accs:
            ac[pl.ds(i * 16, 16)] = negv
        return c

    lax.fori_loop(0, ACCN // 16, init, 0)

    def start(k, dref, vrefs, sem):
        pltpu.async_copy(dst_hbm.at[pl.ds(ebase + k * EB3, EB3)], dref, sem)
        for j, vr in enumerate(vrefs):
            pltpu.async_copy(
                ht_hbm.at[pl.ds((cp * SPLITF + j) * E + ebase + k * EB3, EB3)],
                vr, sem,
            )

    def drain(dref, vrefs, sem):
        pltpu.make_async_copy(dst_hbm.at[pl.ds(0, EB3)], dref, sem).wait()
        for vr in vrefs:
            pltpu.make_async_copy(ht_hbm.at[pl.ds(0, EB3)], vr, sem).wait()

    def process(dref, vrefs):
        def vec(j, c2):
            idx = dref[pl.ds(j * 16, 16)]
            plsc.store_scatter(mk, [idx], lane)
            g = plsc.load_gather(mk, [idx])
            win = g == lane
            vals = []
            for vr, ac in zip(vrefs, accs):
                val = vr[pl.ds(j * 16, 16)]
                vals.append(val)
                a = plsc.load_gather(ac, [idx])
                plsc.store_scatter(ac, [idx], jnp.maximum(a, val), mask=win)

            def cond(pend):
                return jnp.any(pend)

            def body(pend):
                plsc.store_scatter(mk, [idx], lane, mask=pend)
                g2 = plsc.load_gather(mk, [idx])
                w2 = pend & (g2 == lane)
                for val, ac in zip(vals, accs):
                    a2 = plsc.load_gather(ac, [idx])
                    plsc.store_scatter(
                        ac, [idx], jnp.maximum(a2, val), mask=w2
                    )
                return pend & jnp.logical_not(w2)

            lax.while_loop(cond, body, jnp.logical_not(win))
            return c2

        lax.fori_loop(0, EB3 // 16, vec, 0)

    start(0, dA, vAs, semA)

    def pair(p, carry):
        start(2 * p + 1, dB, vBs, semB)
        drain(dA, vAs, semA)
        process(dA, vAs)
        start(2 * p + 2, dA, vAs, semA)
        drain(dB, vBs, semB)
        process(dB, vBs)
        return carry

    lax.fori_loop(0, NB3 // 2, pair, 0)
    drain(dA, vAs, semA)
    process(dA, vAs)
    for j, ac in enumerate(accs):
        pltpu.sync_copy(
            ac, m_out.at[pl.ds((wid * SPLITF + j) * ACCN, ACCN)]
        )


CB = 1024


def _out_mlp_body(mt_ref, b2_ref, w3_ref, b3_ref, g2_ref, be2_ref,
                  w4_ref, b4_ref, z_ref, st_ref, sc):
    i = pl.program_id(0)
    relu = lambda v: jnp.maximum(v, jnp.float32(0.0))
    m = jnp.max(mt_ref[:, :, :], axis=0).T
    aggr = jnp.where(m > jnp.float32(-1e38), m + b2_ref[:, :], jnp.float32(0.0))
    z1 = relu(
        (jnp.dot(aggr, w3_ref[:, :], preferred_element_type=jnp.float32)
         + b3_ref[:, :]) * g2_ref[:, :] + be2_ref[:, :]
    )
    z = relu(
        jnp.dot(z1, w4_ref[:, :], preferred_element_type=jnp.float32)
        + b4_ref[:, :]
    )
    rows = i * CB + lax.broadcasted_iota(jnp.int32, (CB, 1), 0)
    mask = rows < N
    bm = jnp.max(jnp.where(mask, z, jnp.float32(NEG)))

    @pl.when(i == 0)
    def _():
        sc[0] = jnp.float32(NEG)
        sc[1] = jnp.float32(0.0)

    mold = sc[0]
    mn = jnp.maximum(mold, bm)
    s = sc[1] * jnp.exp(mold - mn) + jnp.sum(
        jnp.where(mask, jnp.exp(z - mn), jnp.float32(0.0))
    )
    sc[0] = mn
    sc[1] = s

    @pl.when(i == (ACCN // CB) - 1)
    def _():
        st_ref[:, :] = jnp.concatenate(
            [jnp.broadcast_to(mn, (1, 1)), jnp.broadcast_to(s, (1, 1))], axis=1
        )

    z_ref[:, :] = z


def _out_mlp(mt, b2, w3, b3, g2v, be2, w4, b4):
    return pl.pallas_call(
        _out_mlp_body,
        grid=(ACCN // CB,),
        in_specs=[
            pl.BlockSpec((NQ, H, CB), lambda i: (0, 0, i)),
            pl.BlockSpec((1, H), lambda i: (0, 0)),
            pl.BlockSpec((H, H), lambda i: (0, 0)),
            pl.BlockSpec((1, H), lambda i: (0, 0)),
            pl.BlockSpec((1, H), lambda i: (0, 0)),
            pl.BlockSpec((1, H), lambda i: (0, 0)),
            pl.BlockSpec((H, OUT), lambda i: (0, 0)),
            pl.BlockSpec((1, OUT), lambda i: (0, 0)),
        ],
        out_specs=[
            pl.BlockSpec((CB, OUT), lambda i: (i, 0)),
            pl.BlockSpec((1, 2), lambda i: (0, 0)),
        ],
        out_shape=[_f32((ACCN, OUT)), _f32((1, 2))],
        scratch_shapes=[pltpu.SMEM((2,), jnp.float32)],
    )(mt, b2, w3, b3, g2v, be2, w4, b4)


def _norm_body(z_ref, st_ref, y_ref):
    st = st_ref[:, :]
    y_ref[:, :] = jnp.exp(z_ref[:, :] - st[0:1, 0:1]) * (
        jnp.float32(1.0) / st[0:1, 1:2]
    )


def _normalize(z10, stats):
    return pl.pallas_call(
        _norm_body,
        grid=(25,),
        in_specs=[
            pl.BlockSpec((400, OUT), lambda i: (i, 0)),
            pl.BlockSpec((1, 2), lambda i: (0, 0)),
        ],
        out_specs=pl.BlockSpec((400, OUT), lambda i: (i, 0)),
        out_shape=_f32((N, OUT)),
    )(z10, stats)


def kernel(x, pos, norm, edge_index, W1, b1, g1, be1, W2, b2, W3, b3, g2,
           be2, W4, b4):
    src = edge_index[0]
    dst = edge_index[1]
    w1s = W1[:D] * g1[None, :]
    cvect = (g1 * b1 + be1)[:, None]
    w1pp = W1[D:] * g1[None, :]

    t6 = jnp.concatenate(
        [pos[:, 0], pos[:, 1], pos[:, 2], norm[:, 0], norm[:, 1], norm[:, 2]]
    )
    xp = jnp.pad(x, ((0, NPAD - N), (0, 0)))
    a2t = _node_fold(xp, w1s, cvect)
    dotsf, a2gf = _edge_front_kernel()(t6, src, dst, a2t.reshape(-1))
    ht = _edge_mlp(a2gf.reshape(NW, E), dotsf.reshape(6, E), w1pp, W2)
    mflat = _segment_max_kernel()(dst, ht.reshape(-1))
    mt = mflat.reshape(NQ, H, ACCN)
    z, stats = _out_mlp(mt, b2[None, :], W3, b3[None, :], g2[None, :],
                        be2[None, :], W4, b4[None, :])
    y = _normalize(z[:N], stats)
    return y.reshape(-1)


# speedup vs baseline: 2.3406x; 1.1118x over previous
import functools

import jax
import jax.numpy as jnp
import numpy as np
from jax import lax
from jax.experimental import pallas as pl
from jax.experimental.pallas import tpu as pltpu
from jax.experimental.pallas import tpu_sc as plsc

N = 10000
E = 320000
D = 128
H = 32
OUT = 128

NC, NS, L = 2, 16, 16
NW = NC * NS
EB = 1280
NBLK = E // EB
KMAX = -(-NBLK // NW)
GS = 128
NBG = EB // GS

ACCN = 10240
SPLITF = 4
NQ = NW // (H // SPLITF)
EQ = E // NQ
EB3 = 3200
NB3 = EQ // EB3

NEG = -3.0e38


@functools.lru_cache(maxsize=None)
def _sc_mesh():
    return plsc.VectorSubcoreMesh(core_axis_name="c", subcore_axis_name="s")


def _f32(shape):
    return jax.ShapeDtypeStruct(shape, jnp.float32)


NPAD = 10240


def _a2_body(x_ref, w_ref, c_ref, o_ref):
    o_ref[:, :] = (
        lax.dot_general(
            w_ref[:, :], x_ref[:, :], (((0,), (1,)), ((), ())),
            preferred_element_type=jnp.float32,
        )
        + c_ref[:, :]
    )


def _node_fold(xp, w1s, cvect):
    return pl.pallas_call(
        _a2_body,
        grid=(10,),
        in_specs=[
            pl.BlockSpec((1024, D), lambda i: (i, 0)),
            pl.BlockSpec((D, H), lambda i: (0, 0)),
            pl.BlockSpec((H, 1), lambda i: (0, 0)),
        ],
        out_specs=pl.BlockSpec((H, 1024), lambda i: (0, i)),
        out_shape=_f32((H, NPAD)),
    )(xp, w1s, cvect)


SB = 6400
NSB = E // SB


@functools.lru_cache(maxsize=None)
def _edge_front_kernel():
    return pl.kernel(
        _edge_front_body,
        mesh=_sc_mesh(),
        out_type=(_f32((6 * E,)), _f32((NW * E,))),
        scratch_types=[
            pltpu.VMEM((6 * N,), jnp.float32),
            pltpu.VMEM((EB,), jnp.int32),
            pltpu.VMEM((EB,), jnp.int32),
            pltpu.VMEM((6 * EB,), jnp.float32),
            pltpu.VMEM((NPAD,), jnp.float32),
            pltpu.VMEM((SB,), jnp.int32),
            pltpu.VMEM((SB,), jnp.int32),
            pltpu.VMEM((SB,), jnp.float32),
            pltpu.VMEM((SB,), jnp.float32),
            pltpu.SemaphoreType.DMA,
            pltpu.SemaphoreType.DMA,
            pltpu.SemaphoreType.DMA,
            pltpu.SemaphoreType.DMA,
            pltpu.SemaphoreType.DMA,
        ],
        compiler_params=pltpu.CompilerParams(needs_layout_passes=False),
    )


def _edge_front_body(t6_hbm, src_hbm, dst_hbm, a2t_hbm,
                     dots_out, a2g_out, t6v, srcv, dstv, dotsv,
                     tabv, sbufA, sbufB, obufA, obufB,
                     semD, semIA, semIB, semOA, semOB):
    wid = lax.axis_index("s") * NC + lax.axis_index("c")
    pltpu.sync_copy(t6_hbm, t6v)

    def dots_drain():
        for f in range(6):
            pltpu.make_async_copy(
                dotsv.at[pl.ds(f * EB, EB)],
                dots_out.at[pl.ds(f * E, EB)], semD,
            ).wait()

    def block(k, carry):
        blk = wid + NW * k

        @pl.when(blk < NBLK)
        def _():
            base = blk * EB
            pltpu.sync_copy(src_hbm.at[pl.ds(base, EB)], srcv)
            pltpu.sync_copy(dst_hbm.at[pl.ds(base, EB)], dstv)

            @pl.when(k > 0)
            def _():
                dots_drain()

            def vec(j, c2):
                s16 = srcv[pl.ds(j * 16, 16)]
                d16 = dstv[pl.ds(j * 16, 16)]
                pjx = plsc.load_gather(t6v, [s16])
                pjy = plsc.load_gather(t6v, [s16 + N])
                pjz = plsc.load_gather(t6v, [s16 + 2 * N])
                njx = plsc.load_gather(t6v, [s16 + 3 * N])
                njy = plsc.load_gather(t6v, [s16 + 4 * N])
                njz = plsc.load_gather(t6v, [s16 + 5 * N])
                pix = plsc.load_gather(t6v, [d16])
                piy = plsc.load_gather(t6v, [d16 + N])
                piz = plsc.load_gather(t6v, [d16 + 2 * N])
                nix = plsc.load_gather(t6v, [d16 + 3 * N])
                niy = plsc.load_gather(t6v, [d16 + 4 * N])
                niz = plsc.load_gather(t6v, [d16 + 5 * N])
                psx = pjx - pix
                psy = pjy - piy
                psz = pjz - piz
                dd = psx * psx + psy * psy + psz * psz
                nips = nix * psx + niy * psy + niz * psz
                njps = njx * psx + njy * psy + njz * psz
                ninj = nix * njx + niy * njy + niz * njz
                nini = nix * nix + niy * niy + niz * niz
                njnj = njx * njx + njy * njy + njz * njz
                for f, v in enumerate((dd, nips, njps, ninj, nini, njnj)):
                    dotsv[pl.ds(f * EB + j * 16, 16)] = v
                return c2

            lax.fori_loop(0, EB // 16, vec, 0)
            for f in range(6):
                pltpu.async_copy(
                    dotsv.at[pl.ds(f * EB, EB)],
                    dots_out.at[pl.ds(f * E + base, EB)], semD,
                )

        return carry

    lax.fori_loop(0, KMAX, block, 0)
    dots_drain()

    pltpu.sync_copy(a2t_hbm.at[pl.ds(wid * NPAD, NPAD)], tabv)

    def start_in(b, sbuf, sem):
        pltpu.async_copy(src_hbm.at[pl.ds(b * SB, SB)], sbuf, sem)

    def drain_in(sbuf, sem):
        pltpu.make_async_copy(src_hbm.at[pl.ds(0, SB)], sbuf, sem).wait()

    def start_out(b, obuf, sem):
        pltpu.async_copy(obuf, a2g_out.at[pl.ds(wid * E + b * SB, SB)], sem)

    def drain_out(obuf, sem):
        pltpu.make_async_copy(
            obuf, a2g_out.at[pl.ds(wid * E, SB)], sem
        ).wait()

    def gather(sbuf, obuf):
        def gvec(j, c2):
            obuf[pl.ds(j * 16, 16)] = plsc.load_gather(
                tabv, [sbuf[pl.ds(j * 16, 16)]]
            )
            return c2

        lax.fori_loop(0, SB // 16, gvec, 0)

    start_in(0, sbufA, semIA)

    def gpair(p, carry):
        start_in(2 * p + 1, sbufB, semIB)
        drain_in(sbufA, semIA)

        @pl.when(p > 0)
        def _():
            drain_out(obufA, semOA)

        gather(sbufA, obufA)
        start_out(2 * p, obufA, semOA)

        @pl.when(p < NSB // 2 - 1)
        def _():
            start_in(2 * p + 2, sbufA, semIA)

        drain_in(sbufB, semIB)

        @pl.when(p > 0)
        def _():
            drain_out(obufB, semOB)

        gather(sbufB, obufB)
        start_out(2 * p + 1, obufB, semOB)
        return carry

    lax.fori_loop(0, NSB // 2, gpair, 0)
    drain_out(obufA, semOA)
    drain_out(obufB, semOB)


def _atan2_poly(y, x):
    ax = jnp.abs(x)
    hi = jnp.maximum(y, ax)
    lo = jnp.minimum(y, ax)
    r = lo / jnp.maximum(hi, jnp.float32(1e-37))
    s = r * r
    p = jnp.float32(2.90188402868554315e-03)
    p = p * s - jnp.float32(1.62980136087791103e-02)
    p = p * s + jnp.float32(4.30512745506233115e-02)
    p = p * s - jnp.float32(7.53037674892936771e-02)
    p = p * s + jnp.float32(1.06554379229240167e-01)
    p = p * s - jnp.float32(1.42077862045390988e-01)
    p = p * s + jnp.float32(1.99926389418588319e-01)
    p = p * s - jnp.float32(3.33331017604993073e-01)
    a = r + r * s * p
    a = jnp.where(y > ax, jnp.float32(np.pi / 2) - a, a)
    a = jnp.where(x < 0, jnp.float32(np.pi) - a, a)
    return jnp.where((y == 0) & (x == 0), jnp.float32(0.0), a)


EB2 = 6400


def _edge_mlp_body(a2gt_ref, dots_ref, w1pp_ref, w2_ref, o_ref):
    dots = dots_ref[:, :]
    dd = dots[0:1, :]
    nips = dots[1:2, :]
    njps = dots[2:3, :]
    ninj = dots[3:4, :]
    nini = dots[4:5, :]
    njnj = dots[5:6, :]
    relu = lambda v: jnp.maximum(v, jnp.float32(0.0))
    d = jnp.sqrt(dd)
    a1 = _atan2_poly(jnp.sqrt(relu(nini * dd - nips * nips)), nips)
    a2 = _atan2_poly(jnp.sqrt(relu(njnj * dd - njps * njps)), njps)
    a3 = _atan2_poly(jnp.sqrt(relu(nini * njnj - ninj * ninj)), ninj)
    ppf = jnp.concatenate([d, a1, a2, a3], axis=0)
    pre = a2gt_ref[:, :] + lax.dot_general(
        w1pp_ref[:, :], ppf, (((0,), (0,)), ((), ())),
        preferred_element_type=jnp.float32,
    )
    t = relu(pre)
    o_ref[:, :] = lax.dot_general(
        w2_ref[:, :], t, (((0,), (0,)), ((), ())),
        preferred_element_type=jnp.float32,
    )


def _edge_mlp(a2gt, dots, w1pp, w2):
    return pl.pallas_call(
        _edge_mlp_body,
        grid=(E // EB2,),
        in_specs=[
            pl.BlockSpec((H, EB2), lambda i: (0, i)),
            pl.BlockSpec((6, EB2), lambda i: (0, i)),
            pl.BlockSpec((4, H), lambda i: (0, 0)),
            pl.BlockSpec((H, H), lambda i: (0, 0)),
        ],
        out_specs=pl.BlockSpec((H, EB2), lambda i: (0, i)),
        out_shape=_f32((H, E)),
    )(a2gt, dots, w1pp, w2)


@functools.lru_cache(maxsize=None)
def _segment_max_kernel():
    return pl.kernel(
        _segment_max_body,
        mesh=_sc_mesh(),
        out_type=_f32((NQ * NW * ACCN,)),
        scratch_types=[
            [pltpu.VMEM((ACCN,), jnp.float32) for _ in range(SPLITF)],
            pltpu.VMEM((ACCN,), jnp.int32),
            pltpu.VMEM((EB3,), jnp.int32),
            [pltpu.VMEM((EB3,), jnp.float32) for _ in range(SPLITF)],
            pltpu.VMEM((EB3,), jnp.int32),
            [pltpu.VMEM((EB3,), jnp.float32) for _ in range(SPLITF)],
            pltpu.SemaphoreType.DMA,
            pltpu.SemaphoreType.DMA,
        ],
        compiler_params=pltpu.CompilerParams(needs_layout_passes=False),
    )


def _segment_max_body(dst_hbm, ht_hbm, m_out, accs, mk, dA, vAs, dB, vBs,
                      semA, semB):
    wid = lax.axis_index("s") * NC + lax.axis_index("c")
    q = wid // (H // SPLITF)
    cp = wid % (H // SPLITF)
    ebase = q * EQ
    lane = lax.iota(jnp.int32, 16)
    negv = jnp.full((16,), NEG, jnp.float32)

    def init(i, c):
        for ac in accs:
            ac[pl.ds(i * 16, 16)] = negv
        return c

    lax.fori_loop(0, ACCN // 16, init, 0)

    def start(k, dref, vrefs, sem):
        pltpu.async_copy(dst_hbm.at[pl.ds(ebase + k * EB3, EB3)], dref, sem)
        for j, vr in enumerate(vrefs):
            pltpu.async_copy(
                ht_hbm.at[pl.ds((cp * SPLITF + j) * E + ebase + k * EB3, EB3)],
                vr, sem,
            )

    def drain(dref, vrefs, sem):
        pltpu.make_async_copy(dst_hbm.at[pl.ds(0, EB3)], dref, sem).wait()
        for vr in vrefs:
            pltpu.make_async_copy(ht_hbm.at[pl.ds(0, EB3)], vr, sem).wait()

    def process(dref, vrefs):
        def vec(j, c2):
            idx = dref[pl.ds(j * 16, 16)]
            plsc.store_scatter(mk, [idx], lane)
            g = plsc.load_gather(mk, [idx])
            win = g == lane
            vals = []
            for vr, ac in zip(vrefs, accs):
                val = vr[pl.ds(j * 16, 16)]
                vals.append(val)
                a = plsc.load_gather(ac, [idx])
                plsc.store_scatter(ac, [idx], jnp.maximum(a, val), mask=win)

            def cond(pend):
                return jnp.any(pend)

            def body(pend):
                plsc.store_scatter(mk, [idx], lane, mask=pend)
                g2 = plsc.load_gather(mk, [idx])
                w2 = pend & (g2 == lane)
                for val, ac in zip(vals, accs):
                    a2 = plsc.load_gather(ac, [idx])
                    plsc.store_scatter(
                        ac, [idx], jnp.maximum(a2, val), mask=w2
                    )
                return pend & jnp.logical_not(w2)

            lax.while_loop(cond, body, jnp.logical_not(win))
            return c2

        lax.fori_loop(0, EB3 // 16, vec, 0)

    start(0, dA, vAs, semA)

    def pair(p, carry):
        start(2 * p + 1, dB, vBs, semB)
        drain(dA, vAs, semA)
        process(dA, vAs)
        start(2 * p + 2, dA, vAs, semA)
        drain(dB, vBs, semB)
        process(dB, vBs)
        return carry

    lax.fori_loop(0, NB3 // 2, pair, 0)
    drain(dA, vAs, semA)
    process(dA, vAs)
    for j, ac in enumerate(accs):
        pltpu.sync_copy(
            ac, m_out.at[pl.ds((wid * SPLITF + j) * ACCN, ACCN)]
        )


CB = 1024


def _out_mlp_body(mt_ref, b2_ref, w3_ref, b3_ref, g2_ref, be2_ref,
                  w4_ref, b4_ref, z_ref, st_ref, sc):
    i = pl.program_id(0)
    relu = lambda v: jnp.maximum(v, jnp.float32(0.0))
    m = jnp.max(mt_ref[:, :, :], axis=0).T
    aggr = jnp.where(m > jnp.float32(-1e38), m + b2_ref[:, :], jnp.float32(0.0))
    z1 = relu(
        (jnp.dot(aggr, w3_ref[:, :], preferred_element_type=jnp.float32)
         + b3_ref[:, :]) * g2_ref[:, :] + be2_ref[:, :]
    )
    z = relu(
        jnp.dot(z1, w4_ref[:, :], preferred_element_type=jnp.float32)
        + b4_ref[:, :]
    )
    rows = i * CB + lax.broadcasted_iota(jnp.int32, (CB, 1), 0)
    mask = rows < N
    bm = jnp.max(jnp.where(mask, z, jnp.float32(NEG)))

    @pl.when(i == 0)
    def _():
        sc[0] = jnp.float32(NEG)
        sc[1] = jnp.float32(0.0)

    mold = sc[0]
    mn = jnp.maximum(mold, bm)
    s = sc[1] * jnp.exp(mold - mn) + jnp.sum(
        jnp.where(mask, jnp.exp(z - mn), jnp.float32(0.0))
    )
    sc[0] = mn
    sc[1] = s

    @pl.when(i == (ACCN // CB) - 1)
    def _():
        st_ref[:, :] = jnp.concatenate(
            [jnp.broadcast_to(mn, (1, 1)), jnp.broadcast_to(s, (1, 1))], axis=1
        )

    z_ref[:, :] = z


def _out_mlp(mt, b2, w3, b3, g2v, be2, w4, b4):
    return pl.pallas_call(
        _out_mlp_body,
        grid=(ACCN // CB,),
        in_specs=[
            pl.BlockSpec((NQ, H, CB), lambda i: (0, 0, i)),
            pl.BlockSpec((1, H), lambda i: (0, 0)),
            pl.BlockSpec((H, H), lambda i: (0, 0)),
            pl.BlockSpec((1, H), lambda i: (0, 0)),
            pl.BlockSpec((1, H), lambda i: (0, 0)),
            pl.BlockSpec((1, H), lambda i: (0, 0)),
            pl.BlockSpec((H, OUT), lambda i: (0, 0)),
            pl.BlockSpec((1, OUT), lambda i: (0, 0)),
        ],
        out_specs=[
            pl.BlockSpec((CB, OUT), lambda i: (i, 0)),
            pl.BlockSpec((1, 2), lambda i: (0, 0)),
        ],
        out_shape=[_f32((ACCN, OUT)), _f32((1, 2))],
        scratch_shapes=[pltpu.SMEM((2,), jnp.float32)],
    )(mt, b2, w3, b3, g2v, be2, w4, b4)


def _norm_body(z_ref, st_ref, y_ref):
    st = st_ref[:, :]
    y_ref[:, :] = jnp.exp(z_ref[:, :] - st[0:1, 0:1]) * (
        jnp.float32(1.0) / st[0:1, 1:2]
    )


def _normalize(z10, stats):
    return pl.pallas_call(
        _norm_body,
        grid=(25,),
        in_specs=[
            pl.BlockSpec((400, OUT), lambda i: (i, 0)),
            pl.BlockSpec((1, 2), lambda i: (0, 0)),
        ],
        out_specs=pl.BlockSpec((400, OUT), lambda i: (i, 0)),
        out_shape=_f32((N, OUT)),
    )(z10, stats)


def kernel(x, pos, norm, edge_index, W1, b1, g1, be1, W2, b2, W3, b3, g2,
           be2, W4, b4):
    src = edge_index[0]
    dst = edge_index[1]
    w1s = W1[:D] * g1[None, :]
    cvect = (g1 * b1 + be1)[:, None]
    w1pp = W1[D:] * g1[None, :]

    t6 = jnp.concatenate(
        [pos[:, 0], pos[:, 1], pos[:, 2], norm[:, 0], norm[:, 1], norm[:, 2]]
    )
    xp = jnp.pad(x, ((0, NPAD - N), (0, 0)))
    a2t = _node_fold(xp, w1s, cvect)
    dotsf, a2gf = _edge_front_kernel()(t6, src, dst, a2t.reshape(-1))
    ht = _edge_mlp(a2gf.reshape(NW, E), dotsf.reshape(6, E), w1pp, W2)
    mflat = _segment_max_kernel()(dst, ht.reshape(-1))
    mt = mflat.reshape(NQ, H, ACCN)
    z, stats = _out_mlp(mt, b2[None, :], W3, b3[None, :], g2[None, :],
                        be2[None, :], W4, b4[None, :])
    y = _normalize(z[:N], stats)
    return y.reshape(-1)
